# Initial kernel scaffold; baseline (speedup 1.0000x reference)
import jax
import jax.numpy as jnp
from jax.experimental import pallas as pl


def kernel(x, W, b):
    raise NotImplementedError("write your pallas kernel here")


# speedup vs baseline: 48.3126x; 48.3126x over previous
import functools

import jax
import jax.numpy as jnp
from jax import lax
from jax.experimental import pallas as pl
from jax.experimental.pallas import tpu as pltpu
from jax.experimental.pallas import tpu_sc as plsc

_B, _C, _H, _W = 64, 3, 384, 384
_BINS = 128
_OUT = 128
_NSLICES = _B * _C
_SLICE = _H * _W
_LANES = 16

_NWORKERS = 32
_SPT = _NSLICES // _NWORKERS
_NCHUNKS = 8
_CHUNK = _SLICE // _NCHUNKS
_GROUPS = _CHUNK // _LANES


def _sc_hist_body(x_hbm, out_hbm, buf0, buf1, hist, outbuf, sem0, sem1):
    ncores = 2
    wid = lax.axis_index("s") * ncores + lax.axis_index("c")

    lane = jnp.arange(_LANES, dtype=jnp.int32)
    ones = jnp.ones((_LANES,), dtype=jnp.float32)
    zeros16 = jnp.zeros((_LANES,), dtype=jnp.float32)

    bufs = (buf0, buf1)
    sems = (sem0, sem1)

    def zero_hist(i, _):
        hist[pl.ds(i * _LANES, _LANES)] = zeros16
        return 0

    def make_chunk_processor(buf):
        def process(i, _):
            v = buf[pl.ds(i * _LANES, _LANES)]
            t = jnp.minimum(v * float(_BINS), float(_BINS - 1))
            t = jnp.maximum(t, 0.0)
            bi = t.astype(jnp.int32)
            addr = bi * _LANES + lane
            plsc.addupdate_scatter(hist, [addr], ones)
            return 0

        return process

    procs = tuple(make_chunk_processor(b) for b in bufs)

    for si in range(_SPT):
        sid = wid * _SPT + si

        lax.fori_loop(0, _BINS, zero_hist, 0)

        copies = [None, None]
        copies[0] = pltpu.async_copy(
            x_hbm.at[sid, pl.ds(0, _CHUNK)], bufs[0], sems[0])
        for c in range(_NCHUNKS):
            cur = c % 2
            if c + 1 < _NCHUNKS:
                nxt = (c + 1) % 2
                copies[nxt] = pltpu.async_copy(
                    x_hbm.at[sid, pl.ds((c + 1) * _CHUNK, _CHUNK)],
                    bufs[nxt], sems[nxt])
            copies[cur].wait()
            lax.fori_loop(0, _GROUPS, procs[cur], 0)

        for g in range(_BINS // _LANES):
            acc = zeros16
            base = (g * _LANES + lane) * _LANES
            for j in range(_LANES):
                acc = acc + plsc.load_gather(hist, [base + j])
            outbuf[pl.ds(g * _LANES, _LANES)] = acc

        pltpu.sync_copy(outbuf, out_hbm.at[sid])


@jax.jit
def _sc_histograms(x2d):
    mesh = plsc.VectorSubcoreMesh(core_axis_name="c", subcore_axis_name="s")
    k = functools.partial(
        pl.kernel,
        mesh=mesh,
        out_type=jax.ShapeDtypeStruct((_NSLICES, _BINS), jnp.float32),
        scratch_types=[
            pltpu.VMEM((_CHUNK,), jnp.float32),
            pltpu.VMEM((_CHUNK,), jnp.float32),
            pltpu.VMEM((_BINS * _LANES,), jnp.float32),
            pltpu.VMEM((_BINS,), jnp.float32),
            pltpu.SemaphoreType.DMA,
            pltpu.SemaphoreType.DMA,
        ],
        compiler_params=pltpu.CompilerParams(needs_layout_passes=False),
    )(_sc_hist_body)
    return k(x2d)


def _linear_body(f_ref, w_ref, b_ref, o_ref):
    feats = f_ref[...] * (1.0 / float(_SLICE))
    acc = lax.dot_general(
        feats, w_ref[...], (((1,), (1,)), ((), ())),
        preferred_element_type=jnp.float32)
    o_ref[...] = jnp.maximum(acc + b_ref[...], 0.0)


@jax.jit
def _linear_relu(counts, W, b):
    return pl.pallas_call(
        _linear_body,
        out_shape=jax.ShapeDtypeStruct((_B, _OUT), jnp.float32),
    )(counts, W, b.reshape(1, _OUT))


def kernel(x, W, b):
    x2d = x.reshape(_NSLICES, _SLICE)
    counts = _sc_histograms(x2d)
    return _linear_relu(counts.reshape(_B, _C * _BINS), W, b)


# speedup vs baseline: 50.3392x; 1.0419x over previous
import functools

import jax
import jax.numpy as jnp
from jax import lax
from jax.experimental import pallas as pl
from jax.experimental.pallas import tpu as pltpu
from jax.experimental.pallas import tpu_sc as plsc

_B, _C, _H, _W = 64, 3, 384, 384
_BINS = 128
_OUT = 128
_NSLICES = _B * _C
_SLICE = _H * _W
_LANES = 16

_NWORKERS = 32
_SPT = _NSLICES // _NWORKERS
_NCHUNKS = 8
_CHUNK = _SLICE // _NCHUNKS
_UNROLL = 8
_GROUPS = _CHUNK // (_LANES * _UNROLL)


def _sc_hist_body(x_hbm, out_hbm, buf0, buf1, hist, outbuf, sem0, sem1):
    ncores = 2
    wid = lax.axis_index("s") * ncores + lax.axis_index("c")

    lane = jnp.arange(_LANES, dtype=jnp.int32)
    ones = jnp.ones((_LANES,), dtype=jnp.float32)
    zeros16 = jnp.zeros((_LANES,), dtype=jnp.float32)

    bufs = (buf0, buf1)
    sems = (sem0, sem1)

    def zero_hist(i, _):
        base = i * (_LANES * 8)
        for u in range(8):
            hist[pl.ds(base + u * _LANES, _LANES)] = zeros16
        return 0

    def make_chunk_processor(buf):
        def process(i, _):
            base = i * (_LANES * _UNROLL)
            for u in range(_UNROLL):
                v = buf[pl.ds(base + u * _LANES, _LANES)]
                t = jnp.minimum(v * float(_BINS), float(_BINS - 1))
                t = jnp.maximum(t, 0.0)
                bi = t.astype(jnp.int32)
                addr = bi * _LANES + lane
                plsc.addupdate_scatter(hist, [addr], ones)
            return 0

        return process

    procs = tuple(make_chunk_processor(b) for b in bufs)

    def start_copy(k, sid, c):
        return pltpu.async_copy(
            x_hbm.at[sid, pl.ds(c * _CHUNK, _CHUNK)], bufs[k], sems[k])

    def wait_copy(k):
        pltpu.make_async_copy(
            x_hbm.at[0, pl.ds(0, _CHUNK)], bufs[k], sems[k]).wait()

    def reduce_group(g, _):
        base = (g * _LANES + lane) * _LANES
        acc = zeros16
        for j in range(_LANES):
            rot = jnp.bitwise_and(lane + j, _LANES - 1)
            acc = acc + plsc.load_gather(hist, [base + rot])
        outbuf[pl.ds(g * _LANES, _LANES)] = acc
        return 0

    for si in range(_SPT):
        sid = wid * _SPT + si

        if si == 0:
            lax.fori_loop(0, _BINS * _LANES // (_LANES * 8), zero_hist, 0)
            start_copy(0, sid, 0)
            start_copy(1, sid, 1)

        def chunk_pair(c2, _):
            c = c2 * 2
            wait_copy(0)
            lax.fori_loop(0, _GROUPS, procs[0], 0)

            @pl.when(c + 2 < _NCHUNKS)
            def _():
                start_copy(0, sid, c + 2)

            wait_copy(1)
            lax.fori_loop(0, _GROUPS, procs[1], 0)

            @pl.when(c + 3 < _NCHUNKS)
            def _():
                start_copy(1, sid, c + 3)

            return 0

        lax.fori_loop(0, _NCHUNKS // 2, chunk_pair, 0)

        if si + 1 < _SPT:
            start_copy(0, sid + 1, 0)
            start_copy(1, sid + 1, 1)

        lax.fori_loop(0, _BINS // _LANES, reduce_group, 0)
        pltpu.sync_copy(outbuf, out_hbm.at[sid])

        if si + 1 < _SPT:
            lax.fori_loop(0, _BINS * _LANES // (_LANES * 8), zero_hist, 0)


@jax.jit
def _sc_histograms(x2d):
    mesh = plsc.VectorSubcoreMesh(core_axis_name="c", subcore_axis_name="s")
    k = functools.partial(
        pl.kernel,
        mesh=mesh,
        out_type=jax.ShapeDtypeStruct((_NSLICES, _BINS), jnp.float32),
        scratch_types=[
            pltpu.VMEM((_CHUNK,), jnp.float32),
            pltpu.VMEM((_CHUNK,), jnp.float32),
            pltpu.VMEM((_BINS * _LANES,), jnp.float32),
            pltpu.VMEM((_BINS,), jnp.float32),
            pltpu.SemaphoreType.DMA,
            pltpu.SemaphoreType.DMA,
        ],
        compiler_params=pltpu.CompilerParams(needs_layout_passes=False),
    )(_sc_hist_body)
    return k(x2d)


def _linear_body(f_ref, w_ref, b_ref, o_ref):
    feats = f_ref[...] * (1.0 / float(_SLICE))
    acc = lax.dot_general(
        feats, w_ref[...], (((1,), (1,)), ((), ())),
        preferred_element_type=jnp.float32)
    o_ref[...] = jnp.maximum(acc + b_ref[...], 0.0)


@jax.jit
def _linear_relu(counts, W, b):
    return pl.pallas_call(
        _linear_body,
        out_shape=jax.ShapeDtypeStruct((_B, _OUT), jnp.float32),
    )(counts, W, b.reshape(1, _OUT))


def kernel(x, W, b):
    x2d = x.reshape(_NSLICES, _SLICE)
    counts = _sc_histograms(x2d)
    return _linear_relu(counts.reshape(_B, _C * _BINS), W, b)


# speedup vs baseline: 140.3937x; 2.7890x over previous
import functools

import jax
import jax.numpy as jnp
from jax import lax
from jax.experimental import pallas as pl
from jax.experimental.pallas import tpu as pltpu
from jax.experimental.pallas import tpu_sc as plsc

_B, _C, _H, _W = 64, 3, 384, 384
_BINS = 128
_OUT = 128
_NSLICES = _B * _C
_SLICE = _H * _W
_LANES = 16

_NWORKERS = 32
_SPT = _NSLICES // _NWORKERS
_NCHUNKS = 8
_CHUNK = _SLICE // _NCHUNKS
_UNROLL = 8
_GROUPS = _CHUNK // (_LANES * _UNROLL)


def _sc_hist_body(x_hbm, out_hbm, buf0, buf1, hist, outbuf, sem0, sem1):
    ncores = 2
    wid = lax.axis_index("s") * ncores + lax.axis_index("c")

    lane = jnp.arange(_LANES, dtype=jnp.int32)
    ones = jnp.ones((_LANES,), dtype=jnp.float32)
    zeros16 = jnp.zeros((_LANES,), dtype=jnp.float32)

    bufs = (buf0, buf1)
    sems = (sem0, sem1)

    def zero_hist(i, _):
        base = i * (_LANES * 8)
        for u in range(8):
            hist[pl.ds(base + u * _LANES, _LANES)] = zeros16
        return 0

    def make_chunk_processor(buf):
        def process(i, _):
            base = i * (_LANES * _UNROLL)
            vs = [buf[pl.ds(base + u * _LANES, _LANES)]
                  for u in range(_UNROLL)]
            addrs = []
            for v in vs:
                t = jnp.minimum(v * float(_BINS), float(_BINS - 1))
                t = jnp.maximum(t, 0.0)
                addrs.append(t.astype(jnp.int32) * _LANES + lane)
            for addr in addrs:
                plsc.addupdate_scatter(hist, [addr], ones)
            return 0

        return process

    procs = tuple(make_chunk_processor(b) for b in bufs)

    def start_copy(k, sid, c):
        return pltpu.async_copy(
            x_hbm.at[sid, pl.ds(c * _CHUNK, _CHUNK)], bufs[k], sems[k])

    def wait_copy(k):
        pltpu.make_async_copy(
            x_hbm.at[0, pl.ds(0, _CHUNK)], bufs[k], sems[k]).wait()

    def reduce_group(g, _):
        base = (g * _LANES + lane) * _LANES
        acc = zeros16
        for j in range(_LANES):
            rot = jnp.bitwise_and(lane + j, _LANES - 1)
            acc = acc + plsc.load_gather(hist, [base + rot])
        outbuf[pl.ds(g * _LANES, _LANES)] = acc
        return 0

    for si in range(_SPT):
        sid = wid * _SPT + si

        if si == 0:
            lax.fori_loop(0, _BINS * _LANES // (_LANES * 8), zero_hist, 0)
            start_copy(0, sid, 0)
            start_copy(1, sid, 1)

        def chunk_pair(c2, _):
            c = c2 * 2
            wait_copy(0)
            lax.fori_loop(0, _GROUPS, procs[0], 0)

            @pl.when(c + 2 < _NCHUNKS)
            def _():
                start_copy(0, sid, c + 2)

            wait_copy(1)
            lax.fori_loop(0, _GROUPS, procs[1], 0)

            @pl.when(c + 3 < _NCHUNKS)
            def _():
                start_copy(1, sid, c + 3)

            return 0

        lax.fori_loop(0, _NCHUNKS // 2, chunk_pair, 0)

        if si + 1 < _SPT:
            start_copy(0, sid + 1, 0)
            start_copy(1, sid + 1, 1)

        lax.fori_loop(0, _BINS // _LANES, reduce_group, 0)
        pltpu.sync_copy(outbuf, out_hbm.at[sid])

        if si + 1 < _SPT:
            lax.fori_loop(0, _BINS * _LANES // (_LANES * 8), zero_hist, 0)


@jax.jit
def _sc_histograms(x2d):
    mesh = plsc.VectorSubcoreMesh(core_axis_name="c", subcore_axis_name="s")
    k = functools.partial(
        pl.kernel,
        mesh=mesh,
        out_type=jax.ShapeDtypeStruct((_NSLICES, _BINS), jnp.float32),
        scratch_types=[
            pltpu.VMEM((_CHUNK,), jnp.float32),
            pltpu.VMEM((_CHUNK,), jnp.float32),
            pltpu.VMEM((_BINS * _LANES,), jnp.float32),
            pltpu.VMEM((_BINS,), jnp.float32),
            pltpu.SemaphoreType.DMA,
            pltpu.SemaphoreType.DMA,
        ],
        compiler_params=pltpu.CompilerParams(needs_layout_passes=False),
    )(_sc_hist_body)
    return k(x2d)


def _linear_body(f_ref, w_ref, b_ref, o_ref):
    feats = f_ref[...] * (1.0 / float(_SLICE))
    acc = lax.dot_general(
        feats, w_ref[...], (((1,), (1,)), ((), ())),
        preferred_element_type=jnp.float32)
    o_ref[...] = jnp.maximum(acc + b_ref[...], 0.0)


@jax.jit
def _linear_relu(counts, W, b):
    return pl.pallas_call(
        _linear_body,
        out_shape=jax.ShapeDtypeStruct((_B, _OUT), jnp.float32),
    )(counts, W, b.reshape(1, _OUT))


def kernel(x, W, b):
    x2d = x.reshape(_NSLICES, _SLICE)
    counts = _sc_histograms(x2d)
    return _linear_relu(counts.reshape(_B, _C * _BINS), W, b)


# speedup vs baseline: 190.4137x; 1.3563x over previous
import functools

import jax
import jax.numpy as jnp
from jax import lax
from jax.experimental import pallas as pl
from jax.experimental.pallas import tpu as pltpu
from jax.experimental.pallas import tpu_sc as plsc

_B, _C, _H, _W = 64, 3, 384, 384
_BINS = 128
_OUT = 128
_NSLICES = _B * _C
_SLICE = _H * _W
_LANES = 16

_NWORKERS = 32
_SPT = _NSLICES // _NWORKERS
_NCHUNKS = 8
_CHUNK = _SLICE // _NCHUNKS
_UNROLL = 16
_GROUPS = _CHUNK // (_LANES * _UNROLL)


def _sc_hist_body(x_hbm, out_hbm, buf0, buf1, hist, outbuf, sem0, sem1):
    ncores = 2
    wid = lax.axis_index("s") * ncores + lax.axis_index("c")

    lane = jnp.arange(_LANES, dtype=jnp.int32)
    ones = jnp.ones((_LANES,), dtype=jnp.float32)
    zeros16 = jnp.zeros((_LANES,), dtype=jnp.float32)

    bufs = (buf0, buf1)
    sems = (sem0, sem1)

    def zero_hist(i, _):
        base = i * (_LANES * 8)
        for u in range(8):
            hist[pl.ds(base + u * _LANES, _LANES)] = zeros16
        return 0

    def _scatter_group(vs):
        addrs = []
        for v in vs:
            t = jnp.minimum(v * float(_BINS), float(_BINS - 1))
            t = jnp.maximum(t, 0.0)
            addrs.append(t.astype(jnp.int32) * _LANES + lane)
        for addr in addrs:
            plsc.addupdate_scatter(hist, [addr], ones)

    def make_chunk_processor(buf):
        def load_group(i):
            base = i * (_LANES * _UNROLL)
            return tuple(buf[pl.ds(base + u * _LANES, _LANES)]
                         for u in range(_UNROLL))

        def process(i, carry):
            nxt = load_group(i + 1)
            _scatter_group(carry)
            return nxt

        def run():
            last = lax.fori_loop(0, _GROUPS - 1, process, load_group(0))
            _scatter_group(last)

        return run

    procs = tuple(make_chunk_processor(b) for b in bufs)

    def start_copy(k, sid, c):
        return pltpu.async_copy(
            x_hbm.at[sid, pl.ds(c * _CHUNK, _CHUNK)], bufs[k], sems[k])

    def wait_copy(k):
        pltpu.make_async_copy(
            x_hbm.at[0, pl.ds(0, _CHUNK)], bufs[k], sems[k]).wait()

    def reduce_group(g, _):
        base = (g * _LANES + lane) * _LANES
        acc = zeros16
        for j in range(_LANES):
            rot = jnp.bitwise_and(lane + j, _LANES - 1)
            acc = acc + plsc.load_gather(hist, [base + rot])
        outbuf[pl.ds(g * _LANES, _LANES)] = acc
        return 0

    for si in range(_SPT):
        sid = wid * _SPT + si

        if si == 0:
            lax.fori_loop(0, _BINS * _LANES // (_LANES * 8), zero_hist, 0)
            start_copy(0, sid, 0)
            start_copy(1, sid, 1)

        def chunk_pair(c2, _):
            c = c2 * 2
            wait_copy(0)
            procs[0]()

            @pl.when(c + 2 < _NCHUNKS)
            def _():
                start_copy(0, sid, c + 2)

            wait_copy(1)
            procs[1]()

            @pl.when(c + 3 < _NCHUNKS)
            def _():
                start_copy(1, sid, c + 3)

            return 0

        lax.fori_loop(0, _NCHUNKS // 2, chunk_pair, 0)

        if si + 1 < _SPT:
            start_copy(0, sid + 1, 0)
            start_copy(1, sid + 1, 1)

        lax.fori_loop(0, _BINS // _LANES, reduce_group, 0)
        pltpu.sync_copy(outbuf, out_hbm.at[sid])

        if si + 1 < _SPT:
            lax.fori_loop(0, _BINS * _LANES // (_LANES * 8), zero_hist, 0)


@jax.jit
def _sc_histograms(x2d):
    mesh = plsc.VectorSubcoreMesh(core_axis_name="c", subcore_axis_name="s")
    k = functools.partial(
        pl.kernel,
        mesh=mesh,
        out_type=jax.ShapeDtypeStruct((_NSLICES, _BINS), jnp.float32),
        scratch_types=[
            pltpu.VMEM((_CHUNK,), jnp.float32),
            pltpu.VMEM((_CHUNK,), jnp.float32),
            pltpu.VMEM((_BINS * _LANES,), jnp.float32),
            pltpu.VMEM((_BINS,), jnp.float32),
            pltpu.SemaphoreType.DMA,
            pltpu.SemaphoreType.DMA,
        ],
        compiler_params=pltpu.CompilerParams(needs_layout_passes=False),
    )(_sc_hist_body)
    return k(x2d)


def _linear_body(f_ref, w_ref, b_ref, o_ref):
    feats = f_ref[...] * (1.0 / float(_SLICE))
    acc = lax.dot_general(
        feats, w_ref[...], (((1,), (1,)), ((), ())),
        preferred_element_type=jnp.float32)
    o_ref[...] = jnp.maximum(acc + b_ref[...], 0.0)


@jax.jit
def _linear_relu(counts, W, b):
    return pl.pallas_call(
        _linear_body,
        out_shape=jax.ShapeDtypeStruct((_B, _OUT), jnp.float32),
    )(counts, W, b.reshape(1, _OUT))


def kernel(x, W, b):
    x2d = x.reshape(_NSLICES, _SLICE)
    counts = _sc_histograms(x2d)
    return _linear_relu(counts.reshape(_B, _C * _BINS), W, b)


# speedup vs baseline: 305.1204x; 1.6024x over previous
import functools

import jax
import jax.numpy as jnp
from jax import lax
from jax.experimental import pallas as pl
from jax.experimental.pallas import tpu as pltpu
from jax.experimental.pallas import tpu_sc as plsc

_B, _C, _H, _W = 64, 3, 384, 384
_BINS = 128
_OUT = 128
_NSLICES = _B * _C
_SLICE = _H * _W
_LANES = 16

_NWORKERS = 32
_SPT = _NSLICES // _NWORKERS
_NCHUNKS = 8
_ROWS = _H // _NCHUNKS
_RVREGS = _W // _LANES
_UNROLL = 12
_GROUPS = _ROWS * 2


def _sc_hist_body(x_hbm, out_hbm, buf0, buf1, hist, outbuf, sem0, sem1):
    ncores = 2
    wid = lax.axis_index("s") * ncores + lax.axis_index("c")

    lane = jnp.arange(_LANES, dtype=jnp.int32)
    ones = jnp.ones((_LANES,), dtype=jnp.float32)
    zeros16 = jnp.zeros((_LANES,), dtype=jnp.float32)

    bufs = (buf0, buf1)
    sems = (sem0, sem1)

    def zero_hist(i, _):
        base = i * (_LANES * 8)
        for u in range(8):
            hist[pl.ds(base + u * _LANES, _LANES)] = zeros16
        return 0

    def _scatter_group(vs):
        addrs = []
        for v in vs:
            t = jnp.minimum(v * float(_BINS), float(_BINS - 1))
            t = jnp.maximum(t, 0.0)
            addrs.append(t.astype(jnp.int32) * _LANES + lane)
        for addr in addrs:
            plsc.addupdate_scatter(hist, [addr], ones)

    def make_chunk_processor(buf):
        def load_group(i):
            r = i // 2
            c0 = (i % 2) * (_UNROLL * _LANES)
            return tuple(buf[r, pl.ds(c0 + u * _LANES, _LANES)]
                         for u in range(_UNROLL))

        def process(i, carry):
            nxt = load_group(i + 1)
            _scatter_group(carry)
            return nxt

        def run():
            last = lax.fori_loop(0, _GROUPS - 1, process, load_group(0))
            _scatter_group(last)

        return run

    procs = tuple(make_chunk_processor(b) for b in bufs)

    def start_copy(k, sid, c):
        b = sid // _C
        ch = sid % _C
        return pltpu.async_copy(
            x_hbm.at[b, ch, pl.ds(c * _ROWS, _ROWS)], bufs[k], sems[k])

    def wait_copy(k):
        pltpu.make_async_copy(
            x_hbm.at[0, 0, pl.ds(0, _ROWS)], bufs[k], sems[k]).wait()

    def reduce_group(g, _):
        base = (g * _LANES + lane) * _LANES
        acc = zeros16
        for j in range(_LANES):
            rot = jnp.bitwise_and(lane + j, _LANES - 1)
            acc = acc + plsc.load_gather(hist, [base + rot])
        outbuf[pl.ds(g * _LANES, _LANES)] = acc
        return 0

    for si in range(_SPT):
        sid = wid * _SPT + si

        if si == 0:
            lax.fori_loop(0, _BINS * _LANES // (_LANES * 8), zero_hist, 0)
            start_copy(0, sid, 0)
            start_copy(1, sid, 1)

        def chunk_pair(c2, _):
            c = c2 * 2
            wait_copy(0)
            procs[0]()

            @pl.when(c + 2 < _NCHUNKS)
            def _():
                start_copy(0, sid, c + 2)

            wait_copy(1)
            procs[1]()

            @pl.when(c + 3 < _NCHUNKS)
            def _():
                start_copy(1, sid, c + 3)

            return 0

        lax.fori_loop(0, _NCHUNKS // 2, chunk_pair, 0)

        if si + 1 < _SPT:
            start_copy(0, sid + 1, 0)
            start_copy(1, sid + 1, 1)

        lax.fori_loop(0, _BINS // _LANES, reduce_group, 0)
        pltpu.sync_copy(outbuf, out_hbm.at[sid])

        if si + 1 < _SPT:
            lax.fori_loop(0, _BINS * _LANES // (_LANES * 8), zero_hist, 0)


@jax.jit
def _sc_histograms(x):
    mesh = plsc.VectorSubcoreMesh(core_axis_name="c", subcore_axis_name="s")
    k = functools.partial(
        pl.kernel,
        mesh=mesh,
        out_type=jax.ShapeDtypeStruct((_NSLICES, _BINS), jnp.float32),
        scratch_types=[
            pltpu.VMEM((_ROWS, _W), jnp.float32),
            pltpu.VMEM((_ROWS, _W), jnp.float32),
            pltpu.VMEM((_BINS * _LANES,), jnp.float32),
            pltpu.VMEM((_BINS,), jnp.float32),
            pltpu.SemaphoreType.DMA,
            pltpu.SemaphoreType.DMA,
        ],
        compiler_params=pltpu.CompilerParams(needs_layout_passes=False),
    )(_sc_hist_body)
    return k(x)


def _linear_body(f_ref, w_ref, b_ref, o_ref):
    feats = f_ref[...] * (1.0 / float(_SLICE))
    acc = lax.dot_general(
        feats, w_ref[...], (((1,), (1,)), ((), ())),
        preferred_element_type=jnp.float32)
    o_ref[...] = jnp.maximum(acc + b_ref[...], 0.0)


@jax.jit
def _linear_relu(counts, W, b):
    return pl.pallas_call(
        _linear_body,
        out_shape=jax.ShapeDtypeStruct((_B, _OUT), jnp.float32),
    )(counts, W, b.reshape(1, _OUT))


def kernel(x, W, b):
    counts = _sc_histograms(x)
    return _linear_relu(counts.reshape(_B, _C * _BINS), W, b)
